# baseline (speedup 1.0000x reference)
import functools

import numpy as np
import jax
import jax.numpy as jnp
from jax import lax
from jax.experimental import pallas as pl
from jax.experimental.pallas import tpu as pltpu
from jax.experimental.pallas import tpu_sc as plsc

_B, _T, _C = 16, 2048, 512
_L = 16


def _static_plan(batch_size, max_len_seq, lanes, seed=0):
    rng = np.random.RandomState(seed)
    min_len_seg = 19
    max_len_seg = 32
    max_num_seg = max_len_seq // min_len_seg + 1
    n = batch_size * max_num_seg
    indices = np.broadcast_to(
        np.arange(max_len_seg * 2)[None, :], (n, max_len_seg * 2))
    scales = rng.rand(n) + 0.5
    idx_scaled = indices / scales[:, None]
    idx_scaled_fl = np.floor(idx_scaled)
    lambda_ = idx_scaled - idx_scaled_fl
    len_seg = rng.randint(min_len_seg, max_len_seg, size=(n, 1))
    idx_mask = idx_scaled_fl < (len_seg - 1)
    offset = np.cumsum(len_seg.reshape(batch_size, -1), axis=-1)
    offset = np.pad(offset[:, :-1], ((0, 0), (1, 0)),
                    constant_values=0).reshape(-1, 1)
    idx_scaled_org = idx_scaled_fl + offset
    idx_mask_org = idx_scaled_org < (max_len_seq - 1)
    idx_mask_final = idx_mask & idx_mask_org
    counts = idx_mask_final.sum(axis=-1).reshape(batch_size, -1).sum(axis=-1)
    index_1 = np.repeat(np.arange(batch_size), counts)
    index_2 = idx_scaled_org[idx_mask_final].astype(np.int64)
    lambda_f = lambda_[idx_mask_final]
    starts = np.concatenate([[0], np.cumsum(counts)[:-1]])
    pos = np.arange(index_1.shape[0]) - starts[index_1]
    keep = pos < max_len_seq
    i1, i2, lam, pos = index_1[keep], index_2[keep], lambda_f[keep], pos[keep]

    bt = batch_size * max_len_seq
    g0 = np.zeros(bt, np.int32)
    w0 = np.zeros(bt, np.float32)
    w1 = np.zeros(bt, np.float32)
    flatpos = i1 * max_len_seq + pos
    g0[flatpos] = (i1 * max_len_seq + i2).astype(np.int32)
    w0[flatpos] = (1.0 - lam).astype(np.float32)
    w1[flatpos] = lam.astype(np.float32)
    g1 = g0 + 1
    wrep = np.concatenate(
        [np.repeat(w0[:, None], lanes, axis=1),
         np.repeat(w1[:, None], lanes, axis=1)], axis=1)
    return g0, g1, np.ascontiguousarray(wrep, np.float32)


def _make_sc_call(n_sc, row_off):
    try:
        info = plsc.get_sparse_core_info()
        nc, ns = info.num_cores, info.num_subcores
    except Exception:
        nc, ns = 2, 16
    nw = nc * ns
    bt = n_sc
    rows_per_worker = bt // nw
    K = 32
    nchunk = rows_per_worker // K
    nstep = nchunk // 2
    assert rows_per_worker % (2 * K) == 0
    mesh = plsc.VectorSubcoreMesh(core_axis_name="c", subcore_axis_name="s")

    @functools.partial(
        pl.kernel,
        mesh=mesh,
        out_type=jax.ShapeDtypeStruct((_B * _T, _C), jnp.float32),
        scratch_types=[
            pltpu.VMEM((rows_per_worker,), jnp.int32),
            pltpu.VMEM((rows_per_worker,), jnp.int32),
            pltpu.VMEM((2, K, 2 * _L), jnp.float32),
            pltpu.VMEM((2, K, _C), jnp.float32),
            pltpu.VMEM((2, K, _C), jnp.float32),
            pltpu.VMEM((2, K, _C), jnp.float32),
            pltpu.SemaphoreType.DMA,
            pltpu.SemaphoreType.DMA,
            pltpu.SemaphoreType.DMA,
            pltpu.SemaphoreType.DMA,
            pltpu.SemaphoreType.DMA,
            pltpu.SemaphoreType.DMA,
        ],
    )
    def sc_call(x_hbm, g0_hbm, g1_hbm, w_hbm, out_hbm,
                g0_v, g1_v, w_v, a_v, b_v, o_v,
                sa0, sb0, sa1, sb1, so0, so1):
        wid = lax.axis_index("s") * nc + lax.axis_index("c")
        base = wid * rows_per_worker
        pltpu.sync_copy(g0_hbm.at[pl.ds(base, rows_per_worker)], g0_v)
        pltpu.sync_copy(g1_hbm.at[pl.ds(base, rows_per_worker)], g1_v)

        def gather(off, s, sem_a, sem_b):
            pltpu.async_copy(x_hbm.at[g0_v.at[pl.ds(off, K)]], a_v.at[s],
                             sem_a)
            pltpu.async_copy(x_hbm.at[g1_v.at[pl.ds(off, K)]], b_v.at[s],
                             sem_b)

        def drain(dst, sem):
            pltpu.make_async_copy(x_hbm.at[pl.ds(0, K)], dst, sem).wait()

        def drain_store(s, sem):
            pltpu.make_async_copy(o_v.at[s], out_hbm.at[pl.ds(0, K)],
                                  sem).wait()

        def compute(off, s, sem_a, sem_b, sem_o, first):
            pltpu.sync_copy(w_hbm.at[pl.ds(base + off, K)], w_v.at[s])
            drain(a_v.at[s], sem_a)
            drain(b_v.at[s], sem_b)

            @pl.when(jnp.logical_not(first))
            def _():
                drain_store(s, sem_o)

            def row_body(r, carry):
                w0v = w_v[s, r, pl.ds(0, _L)]
                w1v = w_v[s, r, pl.ds(_L, _L)]
                for c in range(_C // _L):
                    av = a_v[s, r, pl.ds(c * _L, _L)]
                    bv = b_v[s, r, pl.ds(c * _L, _L)]
                    o_v[s, r, pl.ds(c * _L, _L)] = w0v * av + w1v * bv
                return carry

            lax.fori_loop(0, K, row_body, 0)
            pltpu.async_copy(o_v.at[s],
                             out_hbm.at[pl.ds(row_off + base + off, K)],
                             sem_o)

        gather(0, 0, sa0, sb0)

        def step(j, carry):
            off0 = 2 * j * K
            off1 = off0 + K
            gather(off1, 1, sa1, sb1)
            compute(off0, 0, sa0, sb0, so0, j == 0)

            @pl.when(j < nstep - 1)
            def _():
                gather(off0 + 2 * K, 0, sa0, sb0)
            compute(off1, 1, sa1, sb1, so1, j == 0)
            return carry

        lax.fori_loop(0, nstep, step, 0)
        drain_store(0, so0)
        drain_store(1, so1)

    return sc_call


_G0, _G1, _WREP = _static_plan(_B, _T, _L)


def _tc_plan(g0, wrep, row_start, nrows, P):
    bt = g0.shape[0]
    nblk = nrows // P
    w0 = wrep[:, 0]
    w1 = wrep[:, _L]
    spans = []
    smins = []
    for i in range(nblk):
        lo = row_start + i * P
        rows = g0[lo:lo + P]
        real = rows[(w0[lo:lo + P] + w1[lo:lo + P]) > 0]
        if len(real) == 0:
            smins.append(0)
            spans.append(2)
        else:
            smin = (int(real.min()) // 8) * 8
            smins.append(smin)
            spans.append(int(real.max()) + 2 - smin)
    span = -(-max(spans) // 8) * 8
    s0 = np.minimum(np.array(smins, np.int32), bt - span).astype(np.int32)
    meta = np.zeros((nblk, P, 8), np.float32)
    for i in range(nblk):
        lo = row_start + i * P
        real = (w0[lo:lo + P] + w1[lo:lo + P]) > 0
        loc = np.where(real, g0[lo:lo + P] - s0[i], 0)
        meta[i, :, 0] = loc.astype(np.float32)
        meta[i, :, 1] = w0[lo:lo + P]
        meta[i, :, 2] = w1[lo:lo + P]
    return s0, meta, span


def _make_tc_call(row_start, nrows, P=256):
    s0_np, meta_np, span = _tc_plan(_G0, _WREP, row_start, nrows, P)
    nblk = nrows // P

    def tc_kernel(s0_ref, meta_ref, x_ref, o_ref, xbuf, sem):
        i = pl.program_id(0)

        @pl.when(i == 0)
        def _():
            pltpu.make_async_copy(
                x_ref.at[pl.ds(pl.multiple_of(s0_ref[0], 8), span)], xbuf.at[0], sem.at[0]
            ).start()

        par = lax.rem(i, 2)
        pltpu.make_async_copy(
            x_ref.at[pl.ds(pl.multiple_of(s0_ref[i], 8), span)], xbuf.at[par], sem.at[par]
        ).wait()

        @pl.when(i + 1 < nblk)
        def _():
            nxt = lax.rem(i + 1, 2)
            pltpu.make_async_copy(
                x_ref.at[pl.ds(pl.multiple_of(s0_ref[i + 1], 8), span)], xbuf.at[nxt],
                sem.at[nxt]
            ).start()

        meta = meta_ref[0]
        loc = meta[:, 0:1]
        w0 = meta[:, 1:2]
        w1 = meta[:, 2:3]
        ios = lax.broadcasted_iota(jnp.int32, (P, span), 1).astype(jnp.float32)
        W = (jnp.where(ios == loc, w0, 0.0)
             + jnp.where(ios == loc + 1.0, w1, 0.0))
        o_ref[...] = jax.lax.dot(
            W, xbuf[par], precision=jax.lax.Precision.DEFAULT,
            preferred_element_type=jnp.float32)

    call = pl.pallas_call(
        tc_kernel,
        grid=(nblk,),
        in_specs=[
            pl.BlockSpec(memory_space=pltpu.SMEM),
            pl.BlockSpec((1, P, 8), lambda i: (i, 0, 0)),
            pl.BlockSpec(memory_space=pl.ANY),
        ],
        out_specs=pl.BlockSpec((P, _C), lambda i: (i, 0)),
        out_shape=jax.ShapeDtypeStruct((nrows, _C), jnp.float32),
        scratch_shapes=[
            pltpu.VMEM((2, span, _C), jnp.float32),
            pltpu.SemaphoreType.DMA((2,)),
        ],
    )
    def run(xf):
        return call(jnp.asarray(s0_np), jnp.asarray(meta_np), xf)

    return run


_N_SC = 22528
_N_TC = _B * _T - _N_SC

_SC_CALL = _make_sc_call(_N_SC, _N_TC)
_TC_CALL = _make_tc_call(0, _N_TC)


@jax.jit
def kernel(x):
    xf = x.reshape(_B * _T, _C)
    sc_full = _SC_CALL(
        xf, jnp.asarray(_G0[_N_TC:]), jnp.asarray(_G1[_N_TC:]),
        jnp.asarray(_WREP[_N_TC:]))
    tc_part = _TC_CALL(xf)
    out = lax.dynamic_update_slice(sc_full, tc_part, (0, 0))
    return out.reshape(_B, _T, _C)


# speedup vs baseline: 1.0437x; 1.0437x over previous
import functools

import numpy as np
import jax
import jax.numpy as jnp
from jax import lax
from jax.experimental import pallas as pl
from jax.experimental.pallas import tpu as pltpu
from jax.experimental.pallas import tpu_sc as plsc

_B, _T, _C = 16, 2048, 512
_L = 16


def _static_plan(batch_size, max_len_seq, lanes, seed=0):
    rng = np.random.RandomState(seed)
    min_len_seg = 19
    max_len_seg = 32
    max_num_seg = max_len_seq // min_len_seg + 1
    n = batch_size * max_num_seg
    indices = np.broadcast_to(
        np.arange(max_len_seg * 2)[None, :], (n, max_len_seg * 2))
    scales = rng.rand(n) + 0.5
    idx_scaled = indices / scales[:, None]
    idx_scaled_fl = np.floor(idx_scaled)
    lambda_ = idx_scaled - idx_scaled_fl
    len_seg = rng.randint(min_len_seg, max_len_seg, size=(n, 1))
    idx_mask = idx_scaled_fl < (len_seg - 1)
    offset = np.cumsum(len_seg.reshape(batch_size, -1), axis=-1)
    offset = np.pad(offset[:, :-1], ((0, 0), (1, 0)),
                    constant_values=0).reshape(-1, 1)
    idx_scaled_org = idx_scaled_fl + offset
    idx_mask_org = idx_scaled_org < (max_len_seq - 1)
    idx_mask_final = idx_mask & idx_mask_org
    counts = idx_mask_final.sum(axis=-1).reshape(batch_size, -1).sum(axis=-1)
    index_1 = np.repeat(np.arange(batch_size), counts)
    index_2 = idx_scaled_org[idx_mask_final].astype(np.int64)
    lambda_f = lambda_[idx_mask_final]
    starts = np.concatenate([[0], np.cumsum(counts)[:-1]])
    pos = np.arange(index_1.shape[0]) - starts[index_1]
    keep = pos < max_len_seq
    i1, i2, lam, pos = index_1[keep], index_2[keep], lambda_f[keep], pos[keep]

    bt = batch_size * max_len_seq
    g0 = np.zeros(bt, np.int32)
    w0 = np.zeros(bt, np.float32)
    w1 = np.zeros(bt, np.float32)
    flatpos = i1 * max_len_seq + pos
    g0[flatpos] = (i1 * max_len_seq + i2).astype(np.int32)
    w0[flatpos] = (1.0 - lam).astype(np.float32)
    w1[flatpos] = lam.astype(np.float32)
    g1 = g0 + 1
    wrep = np.concatenate(
        [np.repeat(w0[:, None], lanes, axis=1),
         np.repeat(w1[:, None], lanes, axis=1)], axis=1)
    return g0, g1, np.ascontiguousarray(wrep, np.float32)


def _make_sc_call(n_sc, row_off):
    try:
        info = plsc.get_sparse_core_info()
        nc, ns = info.num_cores, info.num_subcores
    except Exception:
        nc, ns = 2, 16
    nw = nc * ns
    bt = n_sc
    rows_per_worker = bt // nw
    K = 32
    nchunk = rows_per_worker // K
    nstep = nchunk // 2
    assert rows_per_worker % (2 * K) == 0
    mesh = plsc.VectorSubcoreMesh(core_axis_name="c", subcore_axis_name="s")

    @functools.partial(
        pl.kernel,
        mesh=mesh,
        out_type=jax.ShapeDtypeStruct((_B * _T, _C), jnp.float32),
        scratch_types=[
            pltpu.VMEM((rows_per_worker,), jnp.int32),
            pltpu.VMEM((rows_per_worker,), jnp.int32),
            pltpu.VMEM((2, K, 2 * _L), jnp.float32),
            pltpu.VMEM((2, K, _C), jnp.float32),
            pltpu.VMEM((2, K, _C), jnp.float32),
            pltpu.VMEM((2, K, _C), jnp.float32),
            pltpu.SemaphoreType.DMA,
            pltpu.SemaphoreType.DMA,
            pltpu.SemaphoreType.DMA,
            pltpu.SemaphoreType.DMA,
            pltpu.SemaphoreType.DMA,
            pltpu.SemaphoreType.DMA,
            pltpu.SemaphoreType.DMA,
            pltpu.SemaphoreType.DMA,
        ],
    )
    def sc_call(x_hbm, g0_hbm, g1_hbm, w_hbm, out_hbm,
                g0_v, g1_v, w_v, a_v, b_v, o_v,
                sa0, sb0, sa1, sb1, so0, so1, sw0, sw1):
        wid = lax.axis_index("s") * nc + lax.axis_index("c")
        base = wid * rows_per_worker
        pltpu.sync_copy(g0_hbm.at[pl.ds(base, rows_per_worker)], g0_v)
        pltpu.sync_copy(g1_hbm.at[pl.ds(base, rows_per_worker)], g1_v)

        def gather(off, s, sem_a, sem_b, sem_w):
            pltpu.async_copy(x_hbm.at[g0_v.at[pl.ds(off, K)]], a_v.at[s],
                             sem_a)
            pltpu.async_copy(x_hbm.at[g1_v.at[pl.ds(off, K)]], b_v.at[s],
                             sem_b)
            pltpu.async_copy(w_hbm.at[pl.ds(base + off, K)], w_v.at[s],
                             sem_w)

        def drain(dst, sem):
            pltpu.make_async_copy(x_hbm.at[pl.ds(0, K)], dst, sem).wait()

        def drain_store(s, sem):
            pltpu.make_async_copy(o_v.at[s], out_hbm.at[pl.ds(0, K)],
                                  sem).wait()

        def compute(off, s, sem_a, sem_b, sem_w, sem_o, first):
            drain(a_v.at[s], sem_a)
            drain(b_v.at[s], sem_b)
            pltpu.make_async_copy(w_hbm.at[pl.ds(0, K)], w_v.at[s],
                                  sem_w).wait()

            @pl.when(jnp.logical_not(first))
            def _():
                drain_store(s, sem_o)

            def row_body(r, carry):
                w0v = w_v[s, r, pl.ds(0, _L)]
                w1v = w_v[s, r, pl.ds(_L, _L)]
                for c in range(_C // _L):
                    av = a_v[s, r, pl.ds(c * _L, _L)]
                    bv = b_v[s, r, pl.ds(c * _L, _L)]
                    o_v[s, r, pl.ds(c * _L, _L)] = w0v * av + w1v * bv
                return carry

            lax.fori_loop(0, K, row_body, 0)
            pltpu.async_copy(o_v.at[s],
                             out_hbm.at[pl.ds(row_off + base + off, K)],
                             sem_o)

        gather(0, 0, sa0, sb0, sw0)

        def step(j, carry):
            off0 = 2 * j * K
            off1 = off0 + K
            gather(off1, 1, sa1, sb1, sw1)
            compute(off0, 0, sa0, sb0, sw0, so0, j == 0)

            @pl.when(j < nstep - 1)
            def _():
                gather(off0 + 2 * K, 0, sa0, sb0, sw0)
            compute(off1, 1, sa1, sb1, sw1, so1, j == 0)
            return carry

        lax.fori_loop(0, nstep, step, 0)
        drain_store(0, so0)
        drain_store(1, so1)

    return sc_call


_G0, _G1, _WREP = _static_plan(_B, _T, _L)


def _tc_plan(g0, wrep, row_start, nrows, P):
    bt = g0.shape[0]
    nblk = nrows // P
    w0 = wrep[:, 0]
    w1 = wrep[:, _L]
    spans = []
    smins = []
    for i in range(nblk):
        lo = row_start + i * P
        rows = g0[lo:lo + P]
        real = rows[(w0[lo:lo + P] + w1[lo:lo + P]) > 0]
        if len(real) == 0:
            smins.append(0)
            spans.append(2)
        else:
            smin = (int(real.min()) // 8) * 8
            smins.append(smin)
            spans.append(int(real.max()) + 2 - smin)
    span = -(-max(spans) // 8) * 8
    s0 = np.minimum(np.array(smins, np.int32), bt - span).astype(np.int32)
    meta = np.zeros((nblk, P, 8), np.float32)
    for i in range(nblk):
        lo = row_start + i * P
        real = (w0[lo:lo + P] + w1[lo:lo + P]) > 0
        loc = np.where(real, g0[lo:lo + P] - s0[i], 0)
        meta[i, :, 0] = loc.astype(np.float32)
        meta[i, :, 1] = w0[lo:lo + P]
        meta[i, :, 2] = w1[lo:lo + P]
    return s0, meta, span


def _make_tc_call(row_start, nrows, P=256):
    s0_np, meta_np, span = _tc_plan(_G0, _WREP, row_start, nrows, P)
    nblk = nrows // P

    def tc_kernel(s0_ref, meta_ref, x_ref, o_ref, xbuf, sem):
        i = pl.program_id(0)

        @pl.when(i == 0)
        def _():
            pltpu.make_async_copy(
                x_ref.at[pl.ds(pl.multiple_of(s0_ref[0], 8), span)], xbuf.at[0], sem.at[0]
            ).start()

        par = lax.rem(i, 2)
        pltpu.make_async_copy(
            x_ref.at[pl.ds(pl.multiple_of(s0_ref[i], 8), span)], xbuf.at[par], sem.at[par]
        ).wait()

        @pl.when(i + 1 < nblk)
        def _():
            nxt = lax.rem(i + 1, 2)
            pltpu.make_async_copy(
                x_ref.at[pl.ds(pl.multiple_of(s0_ref[i + 1], 8), span)], xbuf.at[nxt],
                sem.at[nxt]
            ).start()

        meta = meta_ref[0]
        loc = meta[:, 0:1]
        w0 = meta[:, 1:2]
        w1 = meta[:, 2:3]
        ios = lax.broadcasted_iota(jnp.int32, (P, span), 1).astype(jnp.float32)
        W = (jnp.where(ios == loc, w0, 0.0)
             + jnp.where(ios == loc + 1.0, w1, 0.0))
        o_ref[...] = jax.lax.dot(
            W, xbuf[par], precision=jax.lax.Precision.DEFAULT,
            preferred_element_type=jnp.float32)

    call = pl.pallas_call(
        tc_kernel,
        grid=(nblk,),
        in_specs=[
            pl.BlockSpec(memory_space=pltpu.SMEM),
            pl.BlockSpec((1, P, 8), lambda i: (i, 0, 0)),
            pl.BlockSpec(memory_space=pl.ANY),
        ],
        out_specs=pl.BlockSpec((P, _C), lambda i: (i, 0)),
        out_shape=jax.ShapeDtypeStruct((nrows, _C), jnp.float32),
        scratch_shapes=[
            pltpu.VMEM((2, span, _C), jnp.float32),
            pltpu.SemaphoreType.DMA((2,)),
        ],
    )
    def run(xf):
        return call(jnp.asarray(s0_np), jnp.asarray(meta_np), xf)

    return run


_N_SC = 20480
_N_TC = _B * _T - _N_SC

_SC_CALL = _make_sc_call(_N_SC, _N_TC)
_TC_CALL = _make_tc_call(0, _N_TC)


@jax.jit
def kernel(x):
    xf = x.reshape(_B * _T, _C)
    sc_full = _SC_CALL(
        xf, jnp.asarray(_G0[_N_TC:]), jnp.asarray(_G1[_N_TC:]),
        jnp.asarray(_WREP[_N_TC:]))
    tc_part = _TC_CALL(xf)
    out = lax.dynamic_update_slice(sc_full, tc_part, (0, 0))
    return out.reshape(_B, _T, _C)


# speedup vs baseline: 1.0571x; 1.0129x over previous
import functools

import numpy as np
import jax
import jax.numpy as jnp
from jax import lax
from jax.experimental import pallas as pl
from jax.experimental.pallas import tpu as pltpu
from jax.experimental.pallas import tpu_sc as plsc

_B, _T, _C = 16, 2048, 512
_L = 16


def _static_plan(batch_size, max_len_seq, lanes, seed=0):
    rng = np.random.RandomState(seed)
    min_len_seg = 19
    max_len_seg = 32
    max_num_seg = max_len_seq // min_len_seg + 1
    n = batch_size * max_num_seg
    indices = np.broadcast_to(
        np.arange(max_len_seg * 2)[None, :], (n, max_len_seg * 2))
    scales = rng.rand(n) + 0.5
    idx_scaled = indices / scales[:, None]
    idx_scaled_fl = np.floor(idx_scaled)
    lambda_ = idx_scaled - idx_scaled_fl
    len_seg = rng.randint(min_len_seg, max_len_seg, size=(n, 1))
    idx_mask = idx_scaled_fl < (len_seg - 1)
    offset = np.cumsum(len_seg.reshape(batch_size, -1), axis=-1)
    offset = np.pad(offset[:, :-1], ((0, 0), (1, 0)),
                    constant_values=0).reshape(-1, 1)
    idx_scaled_org = idx_scaled_fl + offset
    idx_mask_org = idx_scaled_org < (max_len_seq - 1)
    idx_mask_final = idx_mask & idx_mask_org
    counts = idx_mask_final.sum(axis=-1).reshape(batch_size, -1).sum(axis=-1)
    index_1 = np.repeat(np.arange(batch_size), counts)
    index_2 = idx_scaled_org[idx_mask_final].astype(np.int64)
    lambda_f = lambda_[idx_mask_final]
    starts = np.concatenate([[0], np.cumsum(counts)[:-1]])
    pos = np.arange(index_1.shape[0]) - starts[index_1]
    keep = pos < max_len_seq
    i1, i2, lam, pos = index_1[keep], index_2[keep], lambda_f[keep], pos[keep]

    bt = batch_size * max_len_seq
    g0 = np.zeros(bt, np.int32)
    w0 = np.zeros(bt, np.float32)
    w1 = np.zeros(bt, np.float32)
    flatpos = i1 * max_len_seq + pos
    g0[flatpos] = (i1 * max_len_seq + i2).astype(np.int32)
    w0[flatpos] = (1.0 - lam).astype(np.float32)
    w1[flatpos] = lam.astype(np.float32)
    g1 = g0 + 1
    wrep = np.concatenate(
        [np.repeat(w0[:, None], lanes, axis=1),
         np.repeat(w1[:, None], lanes, axis=1)], axis=1)
    return g0, g1, np.ascontiguousarray(wrep, np.float32)


def _make_sc_call(n_sc, row_off):
    try:
        info = plsc.get_sparse_core_info()
        nc, ns = info.num_cores, info.num_subcores
    except Exception:
        nc, ns = 2, 16
    nw = nc * ns
    bt = n_sc
    rows_per_worker = bt // nw
    K = 32
    nchunk = rows_per_worker // K
    nstep = nchunk // 2
    assert rows_per_worker % (2 * K) == 0
    mesh = plsc.VectorSubcoreMesh(core_axis_name="c", subcore_axis_name="s")

    @functools.partial(
        pl.kernel,
        mesh=mesh,
        out_type=jax.ShapeDtypeStruct((_B * _T, _C), jnp.float32),
        scratch_types=[
            pltpu.VMEM((rows_per_worker,), jnp.int32),
            pltpu.VMEM((rows_per_worker,), jnp.int32),
            pltpu.VMEM((2, K, 2 * _L), jnp.float32),
            pltpu.VMEM((2, K, _C), jnp.float32),
            pltpu.VMEM((2, K, _C), jnp.float32),
            pltpu.VMEM((2, K, _C), jnp.float32),
            pltpu.SemaphoreType.DMA,
            pltpu.SemaphoreType.DMA,
            pltpu.SemaphoreType.DMA,
            pltpu.SemaphoreType.DMA,
            pltpu.SemaphoreType.DMA,
            pltpu.SemaphoreType.DMA,
            pltpu.SemaphoreType.DMA,
            pltpu.SemaphoreType.DMA,
        ],
    )
    def sc_call(x_hbm, g0_hbm, g1_hbm, w_hbm, out_hbm,
                g0_v, g1_v, w_v, a_v, b_v, o_v,
                sa0, sb0, sa1, sb1, so0, so1, sw0, sw1):
        wid = lax.axis_index("s") * nc + lax.axis_index("c")
        base = wid * rows_per_worker
        pltpu.sync_copy(g0_hbm.at[pl.ds(base, rows_per_worker)], g0_v)
        pltpu.sync_copy(g1_hbm.at[pl.ds(base, rows_per_worker)], g1_v)

        def gather(off, s, sem_a, sem_b, sem_w):
            pltpu.async_copy(x_hbm.at[g0_v.at[pl.ds(off, K)]], a_v.at[s],
                             sem_a)
            pltpu.async_copy(x_hbm.at[g1_v.at[pl.ds(off, K)]], b_v.at[s],
                             sem_b)
            pltpu.async_copy(w_hbm.at[pl.ds(base + off, K)], w_v.at[s],
                             sem_w)

        def drain(dst, sem):
            pltpu.make_async_copy(x_hbm.at[pl.ds(0, K)], dst, sem).wait()

        def drain_store(s, sem):
            pltpu.make_async_copy(o_v.at[s], out_hbm.at[pl.ds(0, K)],
                                  sem).wait()

        def compute(off, s, sem_a, sem_b, sem_w, sem_o, first):
            drain(a_v.at[s], sem_a)
            drain(b_v.at[s], sem_b)
            pltpu.make_async_copy(w_hbm.at[pl.ds(0, K)], w_v.at[s],
                                  sem_w).wait()

            @pl.when(jnp.logical_not(first))
            def _():
                drain_store(s, sem_o)

            def row_body(r, carry):
                w0v = w_v[s, r, pl.ds(0, _L)]
                w1v = w_v[s, r, pl.ds(_L, _L)]
                for c in range(_C // _L):
                    av = a_v[s, r, pl.ds(c * _L, _L)]
                    bv = b_v[s, r, pl.ds(c * _L, _L)]
                    o_v[s, r, pl.ds(c * _L, _L)] = w0v * av + w1v * bv
                return carry

            lax.fori_loop(0, K, row_body, 0)
            pltpu.async_copy(o_v.at[s],
                             out_hbm.at[pl.ds(row_off + base + off, K)],
                             sem_o)

        gather(0, 0, sa0, sb0, sw0)

        def step(j, carry):
            off0 = 2 * j * K
            off1 = off0 + K
            gather(off1, 1, sa1, sb1, sw1)
            compute(off0, 0, sa0, sb0, sw0, so0, j == 0)

            @pl.when(j < nstep - 1)
            def _():
                gather(off0 + 2 * K, 0, sa0, sb0, sw0)
            compute(off1, 1, sa1, sb1, sw1, so1, j == 0)
            return carry

        lax.fori_loop(0, nstep, step, 0)
        drain_store(0, so0)
        drain_store(1, so1)

    return sc_call


_G0, _G1, _WREP = _static_plan(_B, _T, _L)


def _tc_plan(g0, wrep, row_start, nrows, P):
    bt = g0.shape[0]
    nblk = nrows // P
    w0 = wrep[:, 0]
    w1 = wrep[:, _L]
    spans = []
    smins = []
    for i in range(nblk):
        lo = row_start + i * P
        rows = g0[lo:lo + P]
        real = rows[(w0[lo:lo + P] + w1[lo:lo + P]) > 0]
        if len(real) == 0:
            smins.append(0)
            spans.append(2)
        else:
            smin = (int(real.min()) // 8) * 8
            smins.append(smin)
            spans.append(int(real.max()) + 2 - smin)
    span = -(-max(spans) // 8) * 8
    s0 = np.minimum(np.array(smins, np.int32), bt - span).astype(np.int32)
    meta = np.zeros((nblk, P, 8), np.float32)
    for i in range(nblk):
        lo = row_start + i * P
        real = (w0[lo:lo + P] + w1[lo:lo + P]) > 0
        loc = np.where(real, g0[lo:lo + P] - s0[i], 0)
        meta[i, :, 0] = loc.astype(np.float32)
        meta[i, :, 1] = w0[lo:lo + P]
        meta[i, :, 2] = w1[lo:lo + P]
    return s0, meta, span


def _make_tc_call(row_start, nrows, P=256):
    s0_np, meta_np, span = _tc_plan(_G0, _WREP, row_start, nrows, P)
    nblk = nrows // P

    def tc_kernel(s0_ref, meta_ref, x_ref, o_ref, xbuf, sem):
        i = pl.program_id(0)

        @pl.when(i == 0)
        def _():
            pltpu.make_async_copy(
                x_ref.at[pl.ds(pl.multiple_of(s0_ref[0], 8), span)], xbuf.at[0], sem.at[0]
            ).start()

        par = lax.rem(i, 2)
        pltpu.make_async_copy(
            x_ref.at[pl.ds(pl.multiple_of(s0_ref[i], 8), span)], xbuf.at[par], sem.at[par]
        ).wait()

        @pl.when(i + 1 < nblk)
        def _():
            nxt = lax.rem(i + 1, 2)
            pltpu.make_async_copy(
                x_ref.at[pl.ds(pl.multiple_of(s0_ref[i + 1], 8), span)], xbuf.at[nxt],
                sem.at[nxt]
            ).start()

        meta = meta_ref[0]
        loc = meta[:, 0:1]
        w0 = meta[:, 1:2]
        w1 = meta[:, 2:3]
        ios = lax.broadcasted_iota(jnp.int32, (P, span), 1).astype(jnp.float32)
        W = (jnp.where(ios == loc, w0, 0.0)
             + jnp.where(ios == loc + 1.0, w1, 0.0))
        o_ref[...] = jax.lax.dot(
            W, xbuf[par], precision=jax.lax.Precision.DEFAULT,
            preferred_element_type=jnp.float32)

    call = pl.pallas_call(
        tc_kernel,
        grid=(nblk,),
        in_specs=[
            pl.BlockSpec(memory_space=pltpu.SMEM),
            pl.BlockSpec((1, P, 8), lambda i: (i, 0, 0)),
            pl.BlockSpec(memory_space=pl.ANY),
        ],
        out_specs=pl.BlockSpec((P, _C), lambda i: (i, 0)),
        out_shape=jax.ShapeDtypeStruct((nrows, _C), jnp.float32),
        scratch_shapes=[
            pltpu.VMEM((2, span, _C), jnp.float32),
            pltpu.SemaphoreType.DMA((2,)),
        ],
    )
    def run(xf):
        return call(jnp.asarray(s0_np), jnp.asarray(meta_np), xf)

    return run


_N_SC = 20480
_N_TC = _B * _T - _N_SC

_SC_CALL = _make_sc_call(_N_SC, _N_TC)
_TC_CALL = _make_tc_call(0, _N_TC, P=512)


@jax.jit
def kernel(x):
    xf = x.reshape(_B * _T, _C)
    sc_full = _SC_CALL(
        xf, jnp.asarray(_G0[_N_TC:]), jnp.asarray(_G1[_N_TC:]),
        jnp.asarray(_WREP[_N_TC:]))
    tc_part = _TC_CALL(xf)
    out = lax.dynamic_update_slice(sc_full, tc_part, (0, 0))
    return out.reshape(_B, _T, _C)
